# Initial kernel scaffold; baseline (speedup 1.0000x reference)
import jax
import jax.numpy as jnp
from jax.experimental import pallas as pl


def kernel(x, dg_W1, dg_b1, kp_points, kp_W, kp_b, pt_Wq, pt_Wk, pt_Wv, pt_Wp1, pt_bp1, pt_Wp2, pt_bp2, pt_Wg, pt_bg, f_W1, f_b1, f_W2, f_b2):
    raise NotImplementedError("write your pallas kernel here")


# speedup vs baseline: 3.7845x; 3.7845x over previous
import jax
import jax.numpy as jnp
from jax.experimental import pallas as pl
from jax.experimental.pallas import tpu as pltpu

MODEL_DIM = 128
FEATURE_DIM = 256
K_DGCNN = 20
K16 = 16
N_KERNEL = 8
SIGMA = 0.5
P = 256

_HI = jax.lax.Precision.HIGHEST


def _dot(a, b):
    return jax.lax.dot_general(a, b, (((1,), (0,)), ((), ())), precision=_HI)


def _body(x_ref, xT_ref, W1a_ref, W1b_ref, b1_ref, kpT_ref, kn2_ref,
          kpW_ref, kpb_ref, Wq_ref, Wk_ref, Wv_ref, Wp1_ref, bp1_ref,
          Wp2_ref, bp2_ref, Wg_ref, bg_ref, fW1_ref, fb1_ref, fW2_ref,
          fb2_ref, out_ref, acc1_ref, acc2_ref, acc3_ref, dist_ref, xj_ref):
    nb = pl.program_id(1)
    nblocks = pl.num_programs(1)

    x_all = x_ref[0]
    N = x_all.shape[0]
    x_blk = x_ref[0, pl.ds(nb * P, P), :]

    @pl.when(nb == 0)
    def _init():
        acc1_ref[...] = jnp.full((1, MODEL_DIM), -1e30, jnp.float32)
        acc2_ref[...] = jnp.full((1, MODEL_DIM), -1e30, jnp.float32)
        acc3_ref[...] = jnp.full((1, MODEL_DIM), -1e30, jnp.float32)

    dist = jnp.zeros((P, N), jnp.float32)
    for c in range(3):
        d = x_blk[:, c:c + 1] - xT_ref[0, c:c + 1, :]
        dist = dist + d * d
    dist_ref[...] = dist

    def _extract(j, carry):
        dist = dist_ref[...]
        iota = jax.lax.broadcasted_iota(jnp.int32, (P, N), 1)
        m = jnp.min(dist, axis=1, keepdims=True)
        idxv = jnp.min(jnp.where(dist == m, iota, N), axis=1, keepdims=True)
        onehot = (iota == idxv).astype(jnp.float32)
        xj_ref[pl.ds(j * P, P), :] = _dot(onehot, x_all)
        dist_ref[...] = jnp.where(iota == idxv, 1e30, dist)
        return carry

    jax.lax.fori_loop(0, K_DGCNN, _extract, 0)

    xj20 = xj_ref[...]
    xj16 = xj_ref[:K16 * P]
    xrep20 = jnp.concatenate([x_blk] * K_DGCNN, axis=0)
    xrep16 = jnp.concatenate([x_blk] * K16, axis=0)

    base1 = _dot(x_blk, W1b_ref[...]) + b1_ref[...]
    h1 = jax.nn.relu(_dot(xj20 - xrep20, W1a_ref[...]) +
                     jnp.concatenate([base1] * K_DGCNN, axis=0))
    m1 = jnp.full((P, MODEL_DIM), -1e30, jnp.float32)
    for j in range(K_DGCNN):
        m1 = jnp.maximum(m1, h1[j * P:(j + 1) * P])
    acc1_ref[...] = jnp.maximum(acc1_ref[...],
                                jnp.max(m1, axis=0, keepdims=True))

    rel = xj16 - xrep16
    rn2 = jnp.sum(rel * rel, axis=1, keepdims=True)
    dotp = _dot(rel, kpT_ref[...])
    dker = jnp.sqrt(jnp.maximum(rn2 - 2.0 * dotp + kn2_ref[...], 0.0) + 1e-9)
    wgt = jnp.maximum(0.0, 1.0 - dker / SIGMA)
    t2 = jnp.zeros((K16 * P, MODEL_DIM), jnp.float32)
    for K in range(N_KERNEL):
        t2 = t2 + wgt[:, K:K + 1] * _dot(xj16, kpW_ref[K])
    h2 = jnp.zeros((P, MODEL_DIM), jnp.float32)
    for j in range(K16):
        h2 = h2 + t2[j * P:(j + 1) * P]
    h2 = jax.nn.relu(h2 + kpb_ref[...])
    acc2_ref[...] = jnp.maximum(acc2_ref[...],
                                jnp.max(h2, axis=0, keepdims=True))

    q = _dot(x_blk, Wq_ref[...])
    kj = _dot(xj16, Wk_ref[...])
    vj = _dot(xj16, Wv_ref[...])
    pos = xrep16 - xj16
    tp = jax.nn.relu(_dot(pos, Wp1_ref[...]) + bp1_ref[...])
    delta = _dot(tp, Wp2_ref[...]) + bp2_ref[...]
    a = jnp.concatenate([q] * K16, axis=0) - kj + delta
    logit = _dot(a, Wg_ref[...]) + bg_ref[...]
    m3 = jnp.full((P, MODEL_DIM), -1e30, jnp.float32)
    for j in range(K16):
        m3 = jnp.maximum(m3, logit[j * P:(j + 1) * P])
    ssum = jnp.zeros((P, MODEL_DIM), jnp.float32)
    h3 = jnp.zeros((P, MODEL_DIM), jnp.float32)
    for j in range(K16):
        e = jnp.exp(logit[j * P:(j + 1) * P] - m3)
        ssum = ssum + e
        h3 = h3 + e * (vj[j * P:(j + 1) * P] + delta[j * P:(j + 1) * P])
    h3 = h3 / ssum
    acc3_ref[...] = jnp.maximum(acc3_ref[...],
                                jnp.max(h3, axis=0, keepdims=True))

    @pl.when(nb == nblocks - 1)
    def _fuse():
        hf = (_dot(acc1_ref[...], fW1_ref[0]) +
              _dot(acc2_ref[...], fW1_ref[1]) +
              _dot(acc3_ref[...], fW1_ref[2]) + fb1_ref[...])
        hf = jax.nn.relu(hf)
        out_ref[0] = _dot(hf, fW2_ref[...]) + fb2_ref[...]


def kernel(x, dg_W1, dg_b1, kp_points, kp_W, kp_b, pt_Wq, pt_Wk, pt_Wv,
           pt_Wp1, pt_bp1, pt_Wp2, pt_bp2, pt_Wg, pt_bg,
           f_W1, f_b1, f_W2, f_b2):
    B, N, _ = x.shape
    xT = jnp.transpose(x, (0, 2, 1))
    W1a = dg_W1[:3]
    W1b = dg_W1[3:]
    kpT = kp_points.T
    kn2 = jnp.sum(kp_points * kp_points, axis=1)[None, :]
    fW1 = f_W1.reshape(3, MODEL_DIM, FEATURE_DIM)

    full = lambda s: pl.BlockSpec(s, lambda b, nb: tuple(0 for _ in s))
    specs = [
        pl.BlockSpec((1, N, 3), lambda b, nb: (b, 0, 0)),
        pl.BlockSpec((1, 3, N), lambda b, nb: (b, 0, 0)),
        full((3, MODEL_DIM)), full((3, MODEL_DIM)), full((1, MODEL_DIM)),
        full((3, N_KERNEL)), full((1, N_KERNEL)),
        full((N_KERNEL, 3, MODEL_DIM)), full((1, MODEL_DIM)),
        full((3, MODEL_DIM)), full((3, MODEL_DIM)), full((3, MODEL_DIM)),
        full((3, MODEL_DIM)), full((1, MODEL_DIM)),
        full((MODEL_DIM, MODEL_DIM)), full((1, MODEL_DIM)),
        full((MODEL_DIM, MODEL_DIM)), full((1, MODEL_DIM)),
        full((3, MODEL_DIM, FEATURE_DIM)), full((1, FEATURE_DIM)),
        full((FEATURE_DIM, FEATURE_DIM)), full((1, FEATURE_DIM)),
    ]

    out = pl.pallas_call(
        _body,
        grid=(B, N // P),
        in_specs=specs,
        out_specs=pl.BlockSpec((1, 1, FEATURE_DIM), lambda b, nb: (b, 0, 0)),
        out_shape=jax.ShapeDtypeStruct((B, 1, FEATURE_DIM), jnp.float32),
        scratch_shapes=[pltpu.VMEM((1, MODEL_DIM), jnp.float32)] * 3 + [
            pltpu.VMEM((P, N), jnp.float32),
            pltpu.VMEM((K_DGCNN * P, 3), jnp.float32),
        ],
    )(x, xT, W1a, W1b, dg_b1[None, :], kpT, kn2, kp_W, kp_b[None, :],
      pt_Wq, pt_Wk, pt_Wv, pt_Wp1, pt_bp1[None, :], pt_Wp2, pt_bp2[None, :],
      pt_Wg, pt_bg[None, :], fW1, f_b1[None, :], f_W2, f_b2[None, :])
    return out.reshape(B, FEATURE_DIM)


# speedup vs baseline: 6.4656x; 1.7085x over previous
import jax
import jax.numpy as jnp
from jax.experimental import pallas as pl
from jax.experimental.pallas import tpu as pltpu

MODEL_DIM = 128
FEATURE_DIM = 256
K_DGCNN = 20
K16 = 16
N_KERNEL = 8
SIGMA = 0.5
P = 256

_HI = jax.lax.Precision.HIGHEST


def _dot(a, b):
    return jax.lax.dot_general(a, b, (((1,), (0,)), ((), ())), precision=_HI)


def _body(x_ref, xT_ref, W1a_ref, W1b_ref, b1_ref, kpT_ref, kn2_ref,
          kpW_ref, kpb_ref, Wq_ref, Wk_ref, Wv_ref, Wp1_ref, bp1_ref,
          Wp2_ref, bp2_ref, Wg_ref, bg_ref, fW1_ref, fb1_ref, fW2_ref,
          fb2_ref, out_ref, acc1_ref, acc2_ref, acc3_ref, xj_ref):
    nb = pl.program_id(1)
    nblocks = pl.num_programs(1)

    x_all = x_ref[0]
    N = x_all.shape[0]
    x_blk = x_ref[0, pl.ds(nb * P, P), :]

    @pl.when(nb == 0)
    def _init():
        acc1_ref[...] = jnp.full((1, MODEL_DIM), -1e30, jnp.float32)
        acc2_ref[...] = jnp.full((1, MODEL_DIM), -1e30, jnp.float32)
        acc3_ref[...] = jnp.full((1, MODEL_DIM), -1e30, jnp.float32)

    dist = jnp.zeros((P, N), jnp.float32)
    for c in range(3):
        d = x_blk[:, c:c + 1] - xT_ref[0, c:c + 1, :]
        dist = dist + d * d
    def _extract(j, dist):
        iota = jax.lax.broadcasted_iota(jnp.int32, (P, N), 1)
        m = jnp.min(dist, axis=1, keepdims=True)
        idxv = jnp.min(jnp.where(dist == m, iota, N), axis=1, keepdims=True)
        onehot = (iota == idxv).astype(jnp.float32)
        xj_ref[pl.ds(j * P, P), :] = jnp.dot(onehot, x_all)
        return jnp.where(iota == idxv, 1e30, dist)

    jax.lax.fori_loop(0, K_DGCNN, _extract, dist)

    xj20 = xj_ref[...]
    xj16 = xj_ref[:K16 * P]
    xrep20 = jnp.concatenate([x_blk] * K_DGCNN, axis=0)
    xrep16 = jnp.concatenate([x_blk] * K16, axis=0)

    base1 = _dot(x_blk, W1b_ref[...]) + b1_ref[...]
    h1 = jax.nn.relu(_dot(xj20 - xrep20, W1a_ref[...]) +
                     jnp.concatenate([base1] * K_DGCNN, axis=0))
    m1 = jnp.full((P, MODEL_DIM), -1e30, jnp.float32)
    for j in range(K_DGCNN):
        m1 = jnp.maximum(m1, h1[j * P:(j + 1) * P])
    acc1_ref[...] = jnp.maximum(acc1_ref[...],
                                jnp.max(m1, axis=0, keepdims=True))

    rel = xj16 - xrep16
    rn2 = jnp.sum(rel * rel, axis=1, keepdims=True)
    dotp = _dot(rel, kpT_ref[...])
    dker = jnp.sqrt(jnp.maximum(rn2 - 2.0 * dotp + kn2_ref[...], 0.0) + 1e-9)
    wgt = jnp.maximum(0.0, 1.0 - dker / SIGMA)
    t2 = jnp.zeros((K16 * P, MODEL_DIM), jnp.float32)
    for K in range(N_KERNEL):
        t2 = t2 + wgt[:, K:K + 1] * _dot(xj16, kpW_ref[K])
    h2 = jnp.zeros((P, MODEL_DIM), jnp.float32)
    for j in range(K16):
        h2 = h2 + t2[j * P:(j + 1) * P]
    h2 = jax.nn.relu(h2 + kpb_ref[...])
    acc2_ref[...] = jnp.maximum(acc2_ref[...],
                                jnp.max(h2, axis=0, keepdims=True))

    q = _dot(x_blk, Wq_ref[...])
    kj = _dot(xj16, Wk_ref[...])
    vj = _dot(xj16, Wv_ref[...])
    pos = xrep16 - xj16
    tp = jax.nn.relu(_dot(pos, Wp1_ref[...]) + bp1_ref[...])
    delta = _dot(tp, Wp2_ref[...]) + bp2_ref[...]
    a = jnp.concatenate([q] * K16, axis=0) - kj + delta
    logit = _dot(a, Wg_ref[...]) + bg_ref[...]
    m3 = jnp.full((P, MODEL_DIM), -1e30, jnp.float32)
    for j in range(K16):
        m3 = jnp.maximum(m3, logit[j * P:(j + 1) * P])
    ssum = jnp.zeros((P, MODEL_DIM), jnp.float32)
    h3 = jnp.zeros((P, MODEL_DIM), jnp.float32)
    for j in range(K16):
        e = jnp.exp(logit[j * P:(j + 1) * P] - m3)
        ssum = ssum + e
        h3 = h3 + e * (vj[j * P:(j + 1) * P] + delta[j * P:(j + 1) * P])
    h3 = h3 / ssum
    acc3_ref[...] = jnp.maximum(acc3_ref[...],
                                jnp.max(h3, axis=0, keepdims=True))

    @pl.when(nb == nblocks - 1)
    def _fuse():
        hf = (_dot(acc1_ref[...], fW1_ref[0]) +
              _dot(acc2_ref[...], fW1_ref[1]) +
              _dot(acc3_ref[...], fW1_ref[2]) + fb1_ref[...])
        hf = jax.nn.relu(hf)
        out_ref[0] = _dot(hf, fW2_ref[...]) + fb2_ref[...]


def kernel(x, dg_W1, dg_b1, kp_points, kp_W, kp_b, pt_Wq, pt_Wk, pt_Wv,
           pt_Wp1, pt_bp1, pt_Wp2, pt_bp2, pt_Wg, pt_bg,
           f_W1, f_b1, f_W2, f_b2):
    B, N, _ = x.shape
    xT = jnp.transpose(x, (0, 2, 1))
    W1a = dg_W1[:3]
    W1b = dg_W1[3:]
    kpT = kp_points.T
    kn2 = jnp.sum(kp_points * kp_points, axis=1)[None, :]
    fW1 = f_W1.reshape(3, MODEL_DIM, FEATURE_DIM)

    full = lambda s: pl.BlockSpec(s, lambda b, nb: tuple(0 for _ in s))
    specs = [
        pl.BlockSpec((1, N, 3), lambda b, nb: (b, 0, 0)),
        pl.BlockSpec((1, 3, N), lambda b, nb: (b, 0, 0)),
        full((3, MODEL_DIM)), full((3, MODEL_DIM)), full((1, MODEL_DIM)),
        full((3, N_KERNEL)), full((1, N_KERNEL)),
        full((N_KERNEL, 3, MODEL_DIM)), full((1, MODEL_DIM)),
        full((3, MODEL_DIM)), full((3, MODEL_DIM)), full((3, MODEL_DIM)),
        full((3, MODEL_DIM)), full((1, MODEL_DIM)),
        full((MODEL_DIM, MODEL_DIM)), full((1, MODEL_DIM)),
        full((MODEL_DIM, MODEL_DIM)), full((1, MODEL_DIM)),
        full((3, MODEL_DIM, FEATURE_DIM)), full((1, FEATURE_DIM)),
        full((FEATURE_DIM, FEATURE_DIM)), full((1, FEATURE_DIM)),
    ]

    out = pl.pallas_call(
        _body,
        grid=(B, N // P),
        in_specs=specs,
        out_specs=pl.BlockSpec((1, 1, FEATURE_DIM), lambda b, nb: (b, 0, 0)),
        out_shape=jax.ShapeDtypeStruct((B, 1, FEATURE_DIM), jnp.float32),
        scratch_shapes=[pltpu.VMEM((1, MODEL_DIM), jnp.float32)] * 3 + [
            pltpu.VMEM((K_DGCNN * P, 3), jnp.float32),
        ],
    )(x, xT, W1a, W1b, dg_b1[None, :], kpT, kn2, kp_W, kp_b[None, :],
      pt_Wq, pt_Wk, pt_Wv, pt_Wp1, pt_bp1[None, :], pt_Wp2, pt_bp2[None, :],
      pt_Wg, pt_bg[None, :], fW1, f_b1[None, :], f_W2, f_b2[None, :])
    return out.reshape(B, FEATURE_DIM)


# speedup vs baseline: 9.3140x; 1.4406x over previous
import jax
import jax.numpy as jnp
from jax.experimental import pallas as pl
from jax.experimental.pallas import tpu as pltpu

MODEL_DIM = 128
FEATURE_DIM = 256
K_DGCNN = 20
K16 = 16
N_KERNEL = 8
SIGMA = 0.5
P = 256

_HI = jax.lax.Precision.HIGHEST


def _dot(a, b):
    return jax.lax.dot_general(a, b, (((1,), (0,)), ((), ())))


def _body(x_ref, xT_ref, W1a_ref, W1b_ref, b1_ref, kpT_ref, kn2_ref,
          kpW_ref, kpb_ref, Wq_ref, Wk_ref, Wv_ref, Wp1_ref, bp1_ref,
          Wp2_ref, bp2_ref, Wg_ref, bg_ref, fW1_ref, fb1_ref, fW2_ref,
          fb2_ref, out_ref, acc1_ref, acc2_ref, acc3_ref, xj_ref):
    nb = pl.program_id(1)
    nblocks = pl.num_programs(1)

    x_all = x_ref[0]
    N = x_all.shape[0]
    x_blk = x_ref[0, pl.ds(nb * P, P), :]

    @pl.when(nb == 0)
    def _init():
        acc1_ref[...] = jnp.full((1, MODEL_DIM), -1e30, jnp.float32)
        acc2_ref[...] = jnp.full((1, MODEL_DIM), -1e30, jnp.float32)
        acc3_ref[...] = jnp.full((1, MODEL_DIM), -1e30, jnp.float32)

    dist = jnp.zeros((P, N), jnp.float32)
    for c in range(3):
        d = x_blk[:, c:c + 1] - xT_ref[0, c:c + 1, :]
        dist = dist + d * d
    def _extract(j, dist):
        iota = jax.lax.broadcasted_iota(jnp.int32, (P, N), 1)
        m = jnp.min(dist, axis=1, keepdims=True)
        idxv = jnp.min(jnp.where(dist == m, iota, N), axis=1, keepdims=True)
        onehot = (iota == idxv).astype(jnp.float32)
        xj_ref[pl.ds(j * P, P), :] = jnp.dot(onehot, x_all)
        return jnp.where(iota == idxv, 1e30, dist)

    jax.lax.fori_loop(0, K_DGCNN, _extract, dist)

    xj20 = xj_ref[...]
    xj16 = xj_ref[:K16 * P]
    xrep20 = jnp.concatenate([x_blk] * K_DGCNN, axis=0)
    xrep16 = jnp.concatenate([x_blk] * K16, axis=0)

    base1 = _dot(x_blk, W1b_ref[...]) + b1_ref[...]
    h1 = jax.nn.relu(_dot(xj20 - xrep20, W1a_ref[...]) +
                     jnp.concatenate([base1] * K_DGCNN, axis=0))
    m1 = jnp.full((P, MODEL_DIM), -1e30, jnp.float32)
    for j in range(K_DGCNN):
        m1 = jnp.maximum(m1, h1[j * P:(j + 1) * P])
    acc1_ref[...] = jnp.maximum(acc1_ref[...],
                                jnp.max(m1, axis=0, keepdims=True))

    rel = xj16 - xrep16
    rn2 = jnp.sum(rel * rel, axis=1, keepdims=True)
    dotp = _dot(rel, kpT_ref[...])
    dker = jnp.sqrt(jnp.maximum(rn2 - 2.0 * dotp + kn2_ref[...], 0.0) + 1e-9)
    wgt = jnp.maximum(0.0, 1.0 - dker / SIGMA)
    t2 = jnp.zeros((K16 * P, MODEL_DIM), jnp.float32)
    for K in range(N_KERNEL):
        t2 = t2 + wgt[:, K:K + 1] * _dot(xj16, kpW_ref[K])
    h2 = jnp.zeros((P, MODEL_DIM), jnp.float32)
    for j in range(K16):
        h2 = h2 + t2[j * P:(j + 1) * P]
    h2 = jax.nn.relu(h2 + kpb_ref[...])
    acc2_ref[...] = jnp.maximum(acc2_ref[...],
                                jnp.max(h2, axis=0, keepdims=True))

    q = _dot(x_blk, Wq_ref[...])
    kj = _dot(xj16, Wk_ref[...])
    vj = _dot(xj16, Wv_ref[...])
    pos = xrep16 - xj16
    tp = jax.nn.relu(_dot(pos, Wp1_ref[...]) + bp1_ref[...])
    delta = _dot(tp, Wp2_ref[...]) + bp2_ref[...]
    a = jnp.concatenate([q] * K16, axis=0) - kj + delta
    logit = _dot(a, Wg_ref[...]) + bg_ref[...]
    m3 = jnp.full((P, MODEL_DIM), -1e30, jnp.float32)
    for j in range(K16):
        m3 = jnp.maximum(m3, logit[j * P:(j + 1) * P])
    ssum = jnp.zeros((P, MODEL_DIM), jnp.float32)
    h3 = jnp.zeros((P, MODEL_DIM), jnp.float32)
    for j in range(K16):
        e = jnp.exp(logit[j * P:(j + 1) * P] - m3)
        ssum = ssum + e
        h3 = h3 + e * (vj[j * P:(j + 1) * P] + delta[j * P:(j + 1) * P])
    h3 = h3 / ssum
    acc3_ref[...] = jnp.maximum(acc3_ref[...],
                                jnp.max(h3, axis=0, keepdims=True))

    @pl.when(nb == nblocks - 1)
    def _fuse():
        hf = (_dot(acc1_ref[...], fW1_ref[0]) +
              _dot(acc2_ref[...], fW1_ref[1]) +
              _dot(acc3_ref[...], fW1_ref[2]) + fb1_ref[...])
        hf = jax.nn.relu(hf)
        out_ref[0] = _dot(hf, fW2_ref[...]) + fb2_ref[...]


def kernel(x, dg_W1, dg_b1, kp_points, kp_W, kp_b, pt_Wq, pt_Wk, pt_Wv,
           pt_Wp1, pt_bp1, pt_Wp2, pt_bp2, pt_Wg, pt_bg,
           f_W1, f_b1, f_W2, f_b2):
    B, N, _ = x.shape
    xT = jnp.transpose(x, (0, 2, 1))
    W1a = dg_W1[:3]
    W1b = dg_W1[3:]
    kpT = kp_points.T
    kn2 = jnp.sum(kp_points * kp_points, axis=1)[None, :]
    fW1 = f_W1.reshape(3, MODEL_DIM, FEATURE_DIM)

    full = lambda s: pl.BlockSpec(s, lambda b, nb: tuple(0 for _ in s))
    specs = [
        pl.BlockSpec((1, N, 3), lambda b, nb: (b, 0, 0)),
        pl.BlockSpec((1, 3, N), lambda b, nb: (b, 0, 0)),
        full((3, MODEL_DIM)), full((3, MODEL_DIM)), full((1, MODEL_DIM)),
        full((3, N_KERNEL)), full((1, N_KERNEL)),
        full((N_KERNEL, 3, MODEL_DIM)), full((1, MODEL_DIM)),
        full((3, MODEL_DIM)), full((3, MODEL_DIM)), full((3, MODEL_DIM)),
        full((3, MODEL_DIM)), full((1, MODEL_DIM)),
        full((MODEL_DIM, MODEL_DIM)), full((1, MODEL_DIM)),
        full((MODEL_DIM, MODEL_DIM)), full((1, MODEL_DIM)),
        full((3, MODEL_DIM, FEATURE_DIM)), full((1, FEATURE_DIM)),
        full((FEATURE_DIM, FEATURE_DIM)), full((1, FEATURE_DIM)),
    ]

    out = pl.pallas_call(
        _body,
        grid=(B, N // P),
        in_specs=specs,
        out_specs=pl.BlockSpec((1, 1, FEATURE_DIM), lambda b, nb: (b, 0, 0)),
        out_shape=jax.ShapeDtypeStruct((B, 1, FEATURE_DIM), jnp.float32),
        scratch_shapes=[pltpu.VMEM((1, MODEL_DIM), jnp.float32)] * 3 + [
            pltpu.VMEM((K_DGCNN * P, 3), jnp.float32),
        ],
    )(x, xT, W1a, W1b, dg_b1[None, :], kpT, kn2, kp_W, kp_b[None, :],
      pt_Wq, pt_Wk, pt_Wv, pt_Wp1, pt_bp1[None, :], pt_Wp2, pt_bp2[None, :],
      pt_Wg, pt_bg[None, :], fW1, f_b1[None, :], f_W2, f_b2[None, :])
    return out.reshape(B, FEATURE_DIM)
